# Initial kernel scaffold; baseline (speedup 1.0000x reference)
import jax
import jax.numpy as jnp
from jax.experimental import pallas as pl


def kernel(node_feats, edge_feats, params, edge_index):
    raise NotImplementedError("write your pallas kernel here")


# speedup vs baseline: 1.0754x; 1.0754x over previous
import functools

import jax
import jax.numpy as jnp
from jax import lax

HIGHEST = lax.Precision.HIGHEST
from jax.experimental import pallas as pl
from jax.experimental.pallas import tpu as pltpu
from jax.experimental.pallas import tpu_sc as plsc

R = 128
PR = R // 4
V = 6 * PR * PR
E = V * 16
H = 16
C = 128
E_HID = 64
F32 = jnp.float32

NC = 2
NS = 16
NW = NC * NS
EPW = E // NW
CH = 128
NCH = EPW // CH
VPT = V // NS


def _encode_body(x_ref, we_ref, b2d_ref, pw1_ref, pb1_ref, pw2_ref, pb2_ref,
                 out_ref, z_ref):
    f2d = x_ref[0].reshape(R, R * C)
    t1 = jnp.dot(we_ref[...], f2d, preferred_element_type=F32, precision=HIGHEST)
    for i_out in range(PR):
        row = t1[i_out].reshape(R, C)
        z_ref[i_out * PR:(i_out + 1) * PR, :] = jnp.dot(
            we_ref[...], row, preferred_element_type=F32, precision=HIGHEST)
    n = z_ref[...] + b2d_ref[...]
    h1 = jnp.maximum(
        jnp.dot(n, pw1_ref[...], preferred_element_type=F32, precision=HIGHEST) + pb1_ref[...],
        0.0)
    out_ref[0] = jnp.dot(h1, pw2_ref[...], preferred_element_type=F32, precision=HIGHEST) \
        + pb2_ref[...]


def _encode(node_feats, we, b2d_flat, pw1t, pb1, pw2t, pb2):
    return pl.pallas_call(
        _encode_body,
        grid=(6,),
        in_specs=[
            pl.BlockSpec((1, R, R, C), lambda f: (f, 0, 0, 0)),
            pl.BlockSpec((PR, R), lambda f: (0, 0)),
            pl.BlockSpec((PR * PR, 1), lambda f: (0, 0)),
            pl.BlockSpec((C, H), lambda f: (0, 0)),
            pl.BlockSpec((1, H), lambda f: (0, 0)),
            pl.BlockSpec((H, H), lambda f: (0, 0)),
            pl.BlockSpec((1, H), lambda f: (0, 0)),
        ],
        out_specs=pl.BlockSpec((1, PR * PR, H), lambda f: (f, 0, 0)),
        out_shape=jax.ShapeDtypeStruct((6, PR * PR, H), F32),
        scratch_shapes=[pltpu.VMEM((PR * PR, C), F32)],
    )(node_feats, we, b2d_flat, pw1t, pb1, pw2t, pb2)


def _proj_body(x_ref, pw1_ref, pb1_ref, pw2_ref, pb2_ref, out_ref):
    h1 = jnp.maximum(
        jnp.dot(x_ref[...], pw1_ref[...], preferred_element_type=F32, precision=HIGHEST)
        + pb1_ref[...], 0.0)
    out_ref[...] = jnp.dot(h1, pw2_ref[...], preferred_element_type=F32, precision=HIGHEST) \
        + pb2_ref[...]


def _proj(x, pw1t, pb1, pw2t, pb2):
    return pl.pallas_call(
        _proj_body,
        out_shape=jax.ShapeDtypeStruct((V, H), F32),
    )(x, pw1t, pb1, pw2t, pb2)


def _msg_body(ef_ref, g_ref, w1_ref, b1_ref, w2_ref, b2_ref, out_ref):
    e1 = jnp.maximum(
        jnp.dot(ef_ref[...], w1_ref[...], preferred_element_type=F32, precision=HIGHEST)
        + b1_ref[...], 0.0)
    ew = jnp.dot(e1, w2_ref[...], preferred_element_type=F32, precision=HIGHEST) + b2_ref[...]
    g = g_ref[...]
    acc = g[:, 0:1] * ew[:, 0:H]
    for i in range(1, H):
        acc = acc + g[:, i:i + 1] * ew[:, i * H:(i + 1) * H]
    out_ref[...] = acc


def _msg(ef, g, w1t, b1, w2t, b2, blk):
    return pl.pallas_call(
        _msg_body,
        grid=(E // blk,),
        in_specs=[
            pl.BlockSpec((blk, H), lambda e: (e, 0)),
            pl.BlockSpec((blk, H), lambda e: (e, 0)),
            pl.BlockSpec((H, E_HID), lambda e: (0, 0)),
            pl.BlockSpec((1, E_HID), lambda e: (0, 0)),
            pl.BlockSpec((E_HID, H * H), lambda e: (0, 0)),
            pl.BlockSpec((1, H * H), lambda e: (0, 0)),
        ],
        out_specs=pl.BlockSpec((blk, H), lambda e: (e, 0)),
        out_shape=jax.ShapeDtypeStruct((E, H), F32),
    )(ef, g, w1t, b1, w2t, b2)


def _gru_body(p_ref, hid_ref, cb_ref, wih_ref, bih_ref, whh_ref, bhh_ref,
              out_ref):
    node = jnp.maximum(p_ref[0] + p_ref[1] + cb_ref[...], 0.0)
    gi = jnp.dot(node, wih_ref[...], preferred_element_type=F32, precision=HIGHEST) + bih_ref[...]
    gh = jnp.dot(hid_ref[...], whh_ref[...], preferred_element_type=F32, precision=HIGHEST) \
        + bhh_ref[...]
    r = jax.nn.sigmoid(gi[:, 0:H] + gh[:, 0:H])
    z = jax.nn.sigmoid(gi[:, H:2 * H] + gh[:, H:2 * H])
    n = jnp.tanh(gi[:, 2 * H:3 * H] + r * gh[:, 2 * H:3 * H])
    out_ref[...] = (1.0 - z) * n + z * hid_ref[...]


def _gru(parts, hid, cb, wiht, bih, whht, bhh):
    return pl.pallas_call(
        _gru_body,
        out_shape=jax.ShapeDtypeStruct((V, H), F32),
    )(parts, hid, cb, wiht, bih, whht, bhh)


def _dmm_body(x_ref, wt_ref, out_ref):
    out_ref[...] = jnp.dot(x_ref[...], wt_ref[...],
                           preferred_element_type=F32, precision=HIGHEST)


def _dmm(x, wt):
    m = x.shape[0]
    return pl.pallas_call(
        _dmm_body,
        out_shape=jax.ShapeDtypeStruct((m, wt.shape[1]), F32),
    )(x, wt)


def _head_body(z_ref, bd_ref, dw1_ref, db1_ref, dw2_ref, db2_ref, out_ref):
    zin = z_ref[...] + bd_ref[...]
    h1 = jnp.maximum(
        jnp.dot(zin, dw1_ref[...], preferred_element_type=F32, precision=HIGHEST) + db1_ref[...],
        0.0)
    out_ref[...] = jnp.dot(h1, dw2_ref[...], preferred_element_type=F32, precision=HIGHEST) \
        + db2_ref[...]


def _head(z, bd_flat, dw1t, db1, dw2t, db2):
    hblk = 2048
    nb = R * R // hblk
    return pl.pallas_call(
        _head_body,
        grid=(6 * nb,),
        in_specs=[
            pl.BlockSpec((hblk, H), lambda f: (f, 0)),
            pl.BlockSpec((hblk, 1), lambda f: (f % 8, 0)),
            pl.BlockSpec((H, H // 2), lambda f: (0, 0)),
            pl.BlockSpec((1, H // 2), lambda f: (0, 0)),
            pl.BlockSpec((H // 2, 3), lambda f: (0, 0)),
            pl.BlockSpec((1, 3), lambda f: (0, 0)),
        ],
        out_specs=pl.BlockSpec((hblk, 3), lambda f: (f, 0)),
        out_shape=jax.ShapeDtypeStruct((6 * R * R, 3), F32),
    )(z, bd_flat, dw1t, db1, dw2t, db2)


def _decode(xh, wd, bd2d, dw1t, db1, dw2t, db2):
    wdt = wd.T
    xt = xh.reshape(6, PR, PR, H).transpose(0, 3, 1, 2)
    a = _dmm(xt.reshape(6 * H * PR, PR), wdt)
    a = a.reshape(6, H, PR, R).transpose(0, 1, 3, 2)
    b = _dmm(a.reshape(6 * H * R, PR), wdt)
    b = b.reshape(6, H, R, R).transpose(0, 3, 2, 1)
    return _head(b.reshape(6 * R * R, H), bd2d.reshape(R * R, 1),
                 dw1t, db1, dw2t, db2)


@functools.cache
def _sc_mesh():
    return plsc.VectorSubcoreMesh(core_axis_name="c", subcore_axis_name="s",
                                  num_cores=NC, num_subcores=NS)


@functools.cache
def _sc_gather_kernel():
    return pl.kernel(
        _sc_gather_body,
        out_type=jax.ShapeDtypeStruct((E, H), F32),
        mesh=_sc_mesh(),
        scratch_types=[
            pltpu.VMEM((CH,), jnp.int32),
            pltpu.VMEM((CH, H), F32),
            pltpu.SemaphoreType.DMA,
        ],
        compiler_params=pltpu.CompilerParams(use_tc_tiling_on_sc=False),
    )


def _sc_gather(node, src):
    return _sc_gather_kernel()(node, src)


def _sc_gather_body(node_hbm, src_hbm, out_hbm, idx_v, rows_v, sem):
    wid = lax.axis_index("c") * NS + lax.axis_index("s")
    base = wid * EPW

    def body(j, carry):
        off = base + j * CH
        pltpu.sync_copy(src_hbm.at[pl.ds(off, CH)], idx_v)
        pltpu.async_copy(node_hbm.at[idx_v], rows_v, sem).wait()
        pltpu.sync_copy(rows_v, out_hbm.at[pl.ds(off, CH)])
        return carry

    lax.fori_loop(0, NCH, body, 0)


@functools.cache
def _sc_scatter_kernel():
    return pl.kernel(
        _sc_scatter_body,
        out_type=jax.ShapeDtypeStruct((NC, V, H), F32),
        mesh=_sc_mesh(),
        scratch_types=[
            pltpu.VMEM((CH,), jnp.int32),
            pltpu.VMEM((CH, H), F32),
            pltpu.VMEM((VPT, H), F32),
            pltpu.VMEM_SHARED((V, H), F32),
        ],
        compiler_params=pltpu.CompilerParams(use_tc_tiling_on_sc=False),
    )


def _sc_scatter(msg, dst):
    return _sc_scatter_kernel()(msg, dst)


def _sc_scatter_body(msg_hbm, dst_hbm, out_hbm, idx_v, rows_v, stage_v, acc_sh):
    cid = lax.axis_index("c")
    sid = lax.axis_index("s")
    wid = cid * NS + sid
    base = wid * EPW

    def zbody(i, carry):
        stage_v[i, :] = jnp.zeros((H,), F32)
        return carry

    lax.fori_loop(0, VPT, zbody, 0)
    pltpu.sync_copy(stage_v, acc_sh.at[pl.ds(sid * VPT, VPT)])
    plsc.subcore_barrier()

    def body(j, carry):
        off = base + j * CH
        pltpu.sync_copy(dst_hbm.at[pl.ds(off, CH)], idx_v)
        pltpu.sync_copy(msg_hbm.at[pl.ds(off, CH)], rows_v)
        pltpu.sync_copy(rows_v, acc_sh.at[idx_v], add=True)
        return carry

    lax.fori_loop(0, NCH, body, 0)
    plsc.subcore_barrier()

    pltpu.sync_copy(acc_sh.at[pl.ds(sid * VPT, VPT)], stage_v)
    pltpu.sync_copy(stage_v, out_hbm.at[cid, pl.ds(sid * VPT, VPT)])


def _mpnn_block(node, hid, edge_feats, src, dst, p, msg_blk, steps):
    w1t = p['eW1'].T
    b1 = p['eb1'].reshape(1, E_HID)
    w2t = p['eW2'].T
    b2 = p['eb2'].reshape(1, H * H)
    cb = p['cb'].reshape(1, H)
    wiht = p['gWih'].T
    bih = p['gbih'].reshape(1, 3 * H)
    whht = p['gWhh'].T
    bhh = p['gbhh'].reshape(1, 3 * H)
    for _ in range(steps):
        g = _sc_gather(node, src)
        msg = _msg(edge_feats, g, w1t, b1, w2t, b2, msg_blk)
        parts = _sc_scatter(msg, dst)
        hid = _gru(parts, hid, cb, wiht, bih, whht, bhh)
        node = hid
    return hid


def kernel(node_feats, edge_feats, params, edge_index):
    src = edge_index[0]
    dst = edge_index[1]

    we = jnp.matmul(params['encW2'], params['encW1'],
                    precision=HIGHEST)
    be = jnp.matmul(params['encW2'], params['encb1'],
                    precision=HIGHEST) + params['encb2']
    b2d = we.sum(axis=1)[:, None] * be[None, :] + be[:, None]
    wd = jnp.matmul(params['decW2'], params['decW1'],
                    precision=HIGHEST)
    bd = jnp.matmul(params['decW2'], params['decb1'],
                    precision=HIGHEST) + params['decb2']
    bd2d = wd.sum(axis=1)[:, None] * bd[None, :] + bd[:, None]

    pi = params['inp']
    h = _encode(node_feats, we, b2d.reshape(PR * PR, 1),
                pi['pW1'].T, pi['pb1'].reshape(1, H),
                pi['pW2'].T, pi['pb2'].reshape(1, H))
    h = h.reshape(V, H)

    hid = _mpnn_block(h, h, edge_feats, src, dst, pi, 4096, 2)

    pp = params['proc0']
    h2 = _proj(hid, pp['pW1'].T, pp['pb1'].reshape(1, H),
               pp['pW2'].T, pp['pb2'].reshape(1, H))
    hid = _mpnn_block(h2, h2, edge_feats, src, dst, pp, 4096, 2)

    return _decode(hid, wd, bd2d,
                   params['dW1'].T, params['db1'].reshape(1, H // 2),
                   params['dW2'].T, params['db2'].reshape(1, 3))


# speedup vs baseline: 2.3366x; 2.1728x over previous
import functools

import jax
import jax.numpy as jnp
from jax import lax

HIGHEST = lax.Precision.HIGHEST
from jax.experimental import pallas as pl
from jax.experimental.pallas import tpu as pltpu
from jax.experimental.pallas import tpu_sc as plsc

R = 128
PR = R // 4
V = 6 * PR * PR
E = V * 16
H = 16
C = 128
E_HID = 64
F32 = jnp.float32

NC = 2
NS = 16
NW = NC * NS
EPW = E // NW
CH = 128
NCH = EPW // CH
VPT = V // NS


def _encode_body(x_ref, we_ref, b2d_ref, pw1_ref, pb1_ref, pw2_ref, pb2_ref,
                 out_ref, z_ref):
    f2d = x_ref[0].reshape(R, R * C)
    t1 = jnp.dot(we_ref[...], f2d, preferred_element_type=F32, precision=HIGHEST)
    for i_out in range(PR):
        row = t1[i_out].reshape(R, C)
        z_ref[i_out * PR:(i_out + 1) * PR, :] = jnp.dot(
            we_ref[...], row, preferred_element_type=F32, precision=HIGHEST)
    n = z_ref[...] + b2d_ref[...]
    h1 = jnp.maximum(
        jnp.dot(n, pw1_ref[...], preferred_element_type=F32, precision=HIGHEST) + pb1_ref[...],
        0.0)
    out_ref[0] = jnp.dot(h1, pw2_ref[...], preferred_element_type=F32, precision=HIGHEST) \
        + pb2_ref[...]


def _encode(node_feats, we, b2d_flat, pw1t, pb1, pw2t, pb2):
    return pl.pallas_call(
        _encode_body,
        grid=(6,),
        in_specs=[
            pl.BlockSpec((1, R, R, C), lambda f: (f, 0, 0, 0)),
            pl.BlockSpec((PR, R), lambda f: (0, 0)),
            pl.BlockSpec((PR * PR, 1), lambda f: (0, 0)),
            pl.BlockSpec((C, H), lambda f: (0, 0)),
            pl.BlockSpec((1, H), lambda f: (0, 0)),
            pl.BlockSpec((H, H), lambda f: (0, 0)),
            pl.BlockSpec((1, H), lambda f: (0, 0)),
        ],
        out_specs=pl.BlockSpec((1, PR * PR, H), lambda f: (f, 0, 0)),
        out_shape=jax.ShapeDtypeStruct((6, PR * PR, H), F32),
        scratch_shapes=[pltpu.VMEM((PR * PR, C), F32)],
    )(node_feats, we, b2d_flat, pw1t, pb1, pw2t, pb2)


def _proj_body(x_ref, pw1_ref, pb1_ref, pw2_ref, pb2_ref, out_ref):
    h1 = jnp.maximum(
        jnp.dot(pw1_ref[...], x_ref[...], preferred_element_type=F32,
                precision=HIGHEST) + pb1_ref[...], 0.0)
    out_ref[...] = jnp.dot(pw2_ref[...], h1, preferred_element_type=F32,
                           precision=HIGHEST) + pb2_ref[...]


def _proj(xt, pw1, pb1c, pw2, pb2c):
    return pl.pallas_call(
        _proj_body,
        out_shape=jax.ShapeDtypeStruct((H, V), F32),
    )(xt, pw1, pb1c, pw2, pb2c)


def _msg_body(ef_ref, g_ref, w1_ref, b1_ref, w2_ref, b2_ref, out_ref):
    e1 = jnp.maximum(
        jnp.dot(w1_ref[...], ef_ref[...], preferred_element_type=F32,
                precision=HIGHEST) + b1_ref[...], 0.0)
    ew = jnp.dot(w2_ref[...], e1, preferred_element_type=F32,
                 precision=HIGHEST) + b2_ref[...]
    g = g_ref[...]
    acc = g[0:1, :] * ew[0:H, :]
    for i in range(1, H):
        acc = acc + g[i:i + 1, :] * ew[i * H:(i + 1) * H, :]
    out_ref[...] = acc


def _msg(eft, gt, w1, b1c, w2, b2c, blk):
    return pl.pallas_call(
        _msg_body,
        grid=(E // blk,),
        in_specs=[
            pl.BlockSpec((H, blk), lambda e: (0, e)),
            pl.BlockSpec((H, blk), lambda e: (0, e)),
            pl.BlockSpec((E_HID, H), lambda e: (0, 0)),
            pl.BlockSpec((E_HID, 1), lambda e: (0, 0)),
            pl.BlockSpec((H * H, E_HID), lambda e: (0, 0)),
            pl.BlockSpec((H * H, 1), lambda e: (0, 0)),
        ],
        out_specs=pl.BlockSpec((H, blk), lambda e: (0, e)),
        out_shape=jax.ShapeDtypeStruct((H, E), F32),
    )(eft, gt, w1, b1c, w2, b2c)


def _gru_body(p_ref, hid_ref, cb_ref, wih_ref, bih_ref, whh_ref, bhh_ref,
              out_ref):
    node = jnp.maximum(p_ref[0] + p_ref[1] + cb_ref[...], 0.0)
    gi = jnp.dot(wih_ref[...], node, preferred_element_type=F32,
                 precision=HIGHEST) + bih_ref[...]
    gh = jnp.dot(whh_ref[...], hid_ref[...], preferred_element_type=F32,
                 precision=HIGHEST) + bhh_ref[...]
    r = jax.nn.sigmoid(gi[0:H, :] + gh[0:H, :])
    z = jax.nn.sigmoid(gi[H:2 * H, :] + gh[H:2 * H, :])
    n = jnp.tanh(gi[2 * H:3 * H, :] + r * gh[2 * H:3 * H, :])
    out_ref[...] = (1.0 - z) * n + z * hid_ref[...]


def _gru(partst, hidt, cbc, wih, bihc, whh, bhhc):
    return pl.pallas_call(
        _gru_body,
        out_shape=jax.ShapeDtypeStruct((H, V), F32),
    )(partst, hidt, cbc, wih, bihc, whh, bhhc)


def _dmm_body(x_ref, wt_ref, out_ref):
    out_ref[...] = jnp.dot(x_ref[...], wt_ref[...],
                           preferred_element_type=F32, precision=HIGHEST)


def _dmm(x, wt):
    m = x.shape[0]
    return pl.pallas_call(
        _dmm_body,
        out_shape=jax.ShapeDtypeStruct((m, wt.shape[1]), F32),
    )(x, wt)


def _head_body(z_ref, bd_ref, dw1_ref, db1_ref, dw2_ref, db2_ref, out_ref):
    zin = z_ref[...] + bd_ref[...]
    h1 = jnp.maximum(
        jnp.dot(dw1_ref[...], zin, preferred_element_type=F32,
                precision=HIGHEST) + db1_ref[...], 0.0)
    out_ref[...] = jnp.dot(dw2_ref[...], h1, preferred_element_type=F32,
                           precision=HIGHEST) + db2_ref[...]


def _head(zt, bd_row, dw1, db1c, dw2, db2c):
    return pl.pallas_call(
        _head_body,
        grid=(6,),
        in_specs=[
            pl.BlockSpec((H, R * R), lambda f: (0, f)),
            pl.BlockSpec((1, R * R), lambda f: (0, 0)),
            pl.BlockSpec((H // 2, H), lambda f: (0, 0)),
            pl.BlockSpec((H // 2, 1), lambda f: (0, 0)),
            pl.BlockSpec((3, H // 2), lambda f: (0, 0)),
            pl.BlockSpec((3, 1), lambda f: (0, 0)),
        ],
        out_specs=pl.BlockSpec((3, R * R), lambda f: (0, f)),
        out_shape=jax.ShapeDtypeStruct((3, 6 * R * R), F32),
    )(zt, bd_row, dw1, db1c, dw2, db2c)


def _decode(hidt, wd, bd2d, dw1, db1c, dw2, db2c):
    wdt = wd.T
    xt = hidt.reshape(H, 6, PR, PR).transpose(1, 0, 2, 3)
    a = _dmm(xt.reshape(6 * H * PR, PR), wdt)
    a = a.reshape(6, H, PR, R).transpose(0, 1, 3, 2)
    b = _dmm(a.reshape(6 * H * R, PR), wdt)
    zt = b.reshape(6, H, R, R).transpose(1, 0, 3, 2)
    out_t = _head(zt.reshape(H, 6 * R * R), bd2d.reshape(1, R * R),
                  dw1, db1c, dw2, db2c)
    return out_t.T


@functools.cache
def _sc_mesh():
    return plsc.VectorSubcoreMesh(core_axis_name="c", subcore_axis_name="s",
                                  num_cores=NC, num_subcores=NS)


@functools.cache
def _sc_gather_kernel():
    return pl.kernel(
        _sc_gather_body,
        out_type=jax.ShapeDtypeStruct((E, H), F32),
        mesh=_sc_mesh(),
        scratch_types=[
            pltpu.VMEM((CH,), jnp.int32),
            pltpu.VMEM((CH, H), F32),
            pltpu.SemaphoreType.DMA,
        ],
        compiler_params=pltpu.CompilerParams(use_tc_tiling_on_sc=False),
    )


def _sc_gather(node, src):
    return _sc_gather_kernel()(node, src)


def _sc_gather_body(node_hbm, src_hbm, out_hbm, idx_v, rows_v, sem):
    wid = lax.axis_index("c") * NS + lax.axis_index("s")
    base = wid * EPW

    def body(j, carry):
        off = base + j * CH
        pltpu.sync_copy(src_hbm.at[pl.ds(off, CH)], idx_v)
        pltpu.async_copy(node_hbm.at[idx_v], rows_v, sem).wait()
        pltpu.sync_copy(rows_v, out_hbm.at[pl.ds(off, CH)])
        return carry

    lax.fori_loop(0, NCH, body, 0)


@functools.cache
def _sc_scatter_kernel():
    return pl.kernel(
        _sc_scatter_body,
        out_type=jax.ShapeDtypeStruct((NC, V, H), F32),
        mesh=_sc_mesh(),
        scratch_types=[
            pltpu.VMEM((CH,), jnp.int32),
            pltpu.VMEM((CH, H), F32),
            pltpu.VMEM((VPT, H), F32),
            pltpu.VMEM_SHARED((V, H), F32),
        ],
        compiler_params=pltpu.CompilerParams(use_tc_tiling_on_sc=False),
    )


def _sc_scatter(msg, dst):
    return _sc_scatter_kernel()(msg, dst)


def _sc_scatter_body(msg_hbm, dst_hbm, out_hbm, idx_v, rows_v, stage_v, acc_sh):
    cid = lax.axis_index("c")
    sid = lax.axis_index("s")
    wid = cid * NS + sid
    base = wid * EPW

    def zbody(i, carry):
        stage_v[i, :] = jnp.zeros((H,), F32)
        return carry

    lax.fori_loop(0, VPT, zbody, 0)
    pltpu.sync_copy(stage_v, acc_sh.at[pl.ds(sid * VPT, VPT)])
    plsc.subcore_barrier()

    def body(j, carry):
        off = base + j * CH
        pltpu.sync_copy(dst_hbm.at[pl.ds(off, CH)], idx_v)
        pltpu.sync_copy(msg_hbm.at[pl.ds(off, CH)], rows_v)
        pltpu.sync_copy(rows_v, acc_sh.at[idx_v], add=True)
        return carry

    lax.fori_loop(0, NCH, body, 0)
    plsc.subcore_barrier()

    pltpu.sync_copy(acc_sh.at[pl.ds(sid * VPT, VPT)], stage_v)
    pltpu.sync_copy(stage_v, out_hbm.at[cid, pl.ds(sid * VPT, VPT)])


def _mpnn_block(node, hidt, eft, src, dst, p, msg_blk, steps):
    w1 = p['eW1']
    b1c = p['eb1'].reshape(E_HID, 1)
    w2 = p['eW2']
    b2c = p['eb2'].reshape(H * H, 1)
    cbc = p['cb'].reshape(H, 1)
    wih = p['gWih']
    bihc = p['gbih'].reshape(3 * H, 1)
    whh = p['gWhh']
    bhhc = p['gbhh'].reshape(3 * H, 1)
    for _ in range(steps):
        g = _sc_gather(node, src)
        msgt = _msg(eft, g.T, w1, b1c, w2, b2c, msg_blk)
        parts = _sc_scatter(msgt.T, dst)
        hidt = _gru(parts.transpose(0, 2, 1), hidt, cbc, wih, bihc, whh, bhhc)
        node = hidt.T
    return hidt


def kernel(node_feats, edge_feats, params, edge_index):
    src = edge_index[0]
    dst = edge_index[1]

    we = jnp.matmul(params['encW2'], params['encW1'],
                    precision=HIGHEST)
    be = jnp.matmul(params['encW2'], params['encb1'],
                    precision=HIGHEST) + params['encb2']
    b2d = we.sum(axis=1)[:, None] * be[None, :] + be[:, None]
    wd = jnp.matmul(params['decW2'], params['decW1'],
                    precision=HIGHEST)
    bd = jnp.matmul(params['decW2'], params['decb1'],
                    precision=HIGHEST) + params['decb2']
    bd2d = wd.sum(axis=1)[:, None] * bd[None, :] + bd[:, None]

    pi = params['inp']
    h = _encode(node_feats, we, b2d.reshape(PR * PR, 1),
                pi['pW1'].T, pi['pb1'].reshape(1, H),
                pi['pW2'].T, pi['pb2'].reshape(1, H))
    h = h.reshape(V, H)
    eft = edge_feats.T

    hidt = _mpnn_block(h, h.T, eft, src, dst, pi, 8192, 2)

    pp = params['proc0']
    h2t = _proj(hidt, pp['pW1'], pp['pb1'].reshape(H, 1),
                pp['pW2'], pp['pb2'].reshape(H, 1))
    hidt = _mpnn_block(h2t.T, h2t, eft, src, dst, pp, 8192, 2)

    return _decode(hidt, wd, bd2d,
                   params['dW1'], params['db1'].reshape(H // 2, 1),
                   params['dW2'], params['db2'].reshape(3, 1))


# speedup vs baseline: 3.6824x; 1.5760x over previous
import functools

import jax
import jax.numpy as jnp
from jax import lax
from jax.experimental import pallas as pl
from jax.experimental.pallas import tpu as pltpu
from jax.experimental.pallas import tpu_sc as plsc

R = 128
PR = R // 4
V = 6 * PR * PR
E = V * 16
H = 16
C = 128
E_HID = 64
F32 = jnp.float32
BF = jnp.bfloat16

NC = 2
NS = 16
NW = NC * NS
EPW = E // NW
CH = 128
NCH = EPW // CH
VPT = V // NS


def _bdot(a, b):
    return jnp.dot(a.astype(BF), b.astype(BF), preferred_element_type=F32)


def _rnd(x):
    return x.astype(BF).astype(F32)


def _lin2_body(x_ref, w1_ref, b1_ref, w2_ref, b2_ref, out_ref):
    y = _bdot(x_ref[...], w1_ref[...]) + b1_ref[...]
    out_ref[...] = _bdot(y, w2_ref[...]) + b2_ref[...]


def _lin2(x, w1t, b1, w2t, b2, blk):
    m = x.shape[0]
    return pl.pallas_call(
        _lin2_body,
        grid=(m // blk,),
        in_specs=[
            pl.BlockSpec((blk, x.shape[1]), lambda i: (i, 0)),
            pl.BlockSpec(w1t.shape, lambda i: (0, 0)),
            pl.BlockSpec(b1.shape, lambda i: (0, 0)),
            pl.BlockSpec(w2t.shape, lambda i: (0, 0)),
            pl.BlockSpec(b2.shape, lambda i: (0, 0)),
        ],
        out_specs=pl.BlockSpec((blk, w2t.shape[1]), lambda i: (i, 0)),
        out_shape=jax.ShapeDtypeStruct((m, w2t.shape[1]), F32),
    )(x, w1t, b1, w2t, b2)


def _proj_rows_body(x_ref, pw1_ref, pb1_ref, pw2_ref, pb2_ref, out_ref):
    h1 = jnp.maximum(_bdot(x_ref[...], pw1_ref[...]) + pb1_ref[...], 0.0)
    out_ref[...] = _bdot(h1, pw2_ref[...]) + pb2_ref[...]


def _proj_rows(x, pw1t, pb1, pw2t, pb2):
    return pl.pallas_call(
        _proj_rows_body,
        out_shape=jax.ShapeDtypeStruct((V, H), F32),
    )(x, pw1t, pb1, pw2t, pb2)


def _proj_t_body(x_ref, pw1_ref, pb1_ref, pw2_ref, pb2_ref, out_ref):
    h1 = jnp.maximum(_bdot(pw1_ref[...], x_ref[...]) + pb1_ref[...], 0.0)
    out_ref[...] = _bdot(pw2_ref[...], h1) + pb2_ref[...]


def _proj_t(xt, pw1, pb1c, pw2, pb2c):
    return pl.pallas_call(
        _proj_t_body,
        out_shape=jax.ShapeDtypeStruct((H, V), F32),
    )(xt, pw1, pb1c, pw2, pb2c)


def _msg_body(ef_ref, g_ref, w1_ref, b1_ref, w2_ref, b2_ref, out_ref):
    e1 = jnp.maximum(_bdot(w1_ref[...], ef_ref[...]) + b1_ref[...], 0.0)
    ew = _rnd(_bdot(w2_ref[...], e1) + b2_ref[...])
    g = _rnd(g_ref[...])
    acc = g[0:1, :] * ew[0:H, :]
    for i in range(1, H):
        acc = acc + g[i:i + 1, :] * ew[i * H:(i + 1) * H, :]
    out_ref[...] = acc


def _msg(eft, gt, w1, b1c, w2, b2c, blk):
    return pl.pallas_call(
        _msg_body,
        grid=(E // blk,),
        in_specs=[
            pl.BlockSpec((H, blk), lambda e: (0, e)),
            pl.BlockSpec((H, blk), lambda e: (0, e)),
            pl.BlockSpec((E_HID, H), lambda e: (0, 0)),
            pl.BlockSpec((E_HID, 1), lambda e: (0, 0)),
            pl.BlockSpec((H * H, E_HID), lambda e: (0, 0)),
            pl.BlockSpec((H * H, 1), lambda e: (0, 0)),
        ],
        out_specs=pl.BlockSpec((H, blk), lambda e: (0, e)),
        out_shape=jax.ShapeDtypeStruct((H, E), F32),
    )(eft, gt, w1, b1c, w2, b2c)


def _gru_body(p_ref, hid_ref, cb_ref, wih_ref, bih_ref, whh_ref, bhh_ref,
              out_ref):
    node = jnp.maximum(p_ref[0] + p_ref[1] + cb_ref[...], 0.0)
    gi = _bdot(wih_ref[...], node) + bih_ref[...]
    gh = _bdot(whh_ref[...], hid_ref[...]) + bhh_ref[...]
    r = jax.nn.sigmoid(gi[0:H, :] + gh[0:H, :])
    z = jax.nn.sigmoid(gi[H:2 * H, :] + gh[H:2 * H, :])
    n = jnp.tanh(gi[2 * H:3 * H, :] + r * gh[2 * H:3 * H, :])
    out_ref[...] = (1.0 - z) * n + z * hid_ref[...]


def _gru(partst, hidt, cbc, wih, bihc, whh, bhhc):
    return pl.pallas_call(
        _gru_body,
        out_shape=jax.ShapeDtypeStruct((H, V), F32),
    )(partst, hidt, cbc, wih, bihc, whh, bhhc)


def _head_body(z_ref, dw1_ref, db1_ref, dw2_ref, db2_ref, out_ref):
    h1 = jnp.maximum(_bdot(dw1_ref[...], z_ref[...]) + db1_ref[...], 0.0)
    out_ref[...] = _bdot(dw2_ref[...], h1) + db2_ref[...]


def _head(zt, dw1, db1c, dw2, db2c):
    return pl.pallas_call(
        _head_body,
        grid=(6,),
        in_specs=[
            pl.BlockSpec((H, R * R), lambda f: (0, f)),
            pl.BlockSpec((H // 2, H), lambda f: (0, 0)),
            pl.BlockSpec((H // 2, 1), lambda f: (0, 0)),
            pl.BlockSpec((3, H // 2), lambda f: (0, 0)),
            pl.BlockSpec((3, 1), lambda f: (0, 0)),
        ],
        out_specs=pl.BlockSpec((3, R * R), lambda f: (0, f)),
        out_shape=jax.ShapeDtypeStruct((3, 6 * R * R), F32),
    )(zt, dw1, db1c, dw2, db2c)


@functools.cache
def _sc_mesh():
    return plsc.VectorSubcoreMesh(core_axis_name="c", subcore_axis_name="s",
                                  num_cores=NC, num_subcores=NS)


@functools.cache
def _sc_gather_kernel():
    return pl.kernel(
        _sc_gather_body,
        out_type=jax.ShapeDtypeStruct((E, H), F32),
        mesh=_sc_mesh(),
        scratch_types=[
            pltpu.VMEM((NCH, CH), jnp.int32),
            pltpu.VMEM((EPW, H), F32),
            pltpu.SemaphoreType.DMA,
            pltpu.SemaphoreType.DMA,
        ],
        compiler_params=pltpu.CompilerParams(use_tc_tiling_on_sc=False),
    )


def _sc_gather(node, src):
    return _sc_gather_kernel()(node, src)


def _sc_gather_body(node_hbm, src_hbm, out_hbm, idx_v, rows_v, sem_i, sem_g):
    wid = lax.axis_index("c") * NS + lax.axis_index("s")
    base = wid * EPW

    idx_descs = [
        pltpu.async_copy(src_hbm.at[pl.ds(base + j * CH, CH)],
                         idx_v.at[j], sem_i)
        for j in range(NCH)
    ]
    descs = []
    for j in range(NCH):
        idx_descs[j].wait()
        descs.append(
            pltpu.async_copy(node_hbm.at[idx_v.at[j]],
                             rows_v.at[pl.ds(j * CH, CH)], sem_g))
    for d in descs:
        d.wait()
    pltpu.sync_copy(rows_v, out_hbm.at[pl.ds(base, EPW)])


@functools.cache
def _sc_scatter_kernel():
    return pl.kernel(
        _sc_scatter_body,
        out_type=jax.ShapeDtypeStruct((NC, V, H), F32),
        mesh=_sc_mesh(),
        scratch_types=[
            pltpu.VMEM((NCH, CH), jnp.int32),
            pltpu.VMEM((EPW, H), F32),
            pltpu.VMEM((VPT, H), F32),
            pltpu.VMEM_SHARED((V, H), F32),
            pltpu.SemaphoreType.DMA,
            pltpu.SemaphoreType.DMA,
            pltpu.SemaphoreType.DMA,
        ],
        compiler_params=pltpu.CompilerParams(use_tc_tiling_on_sc=False),
    )


def _sc_scatter(msg, dst):
    return _sc_scatter_kernel()(msg, dst)


def _sc_scatter_body(msg_hbm, dst_hbm, out_hbm, idx_v, rows_v, stage_v, acc_sh,
                     sem_i, sem_r, sem_s):
    cid = lax.axis_index("c")
    sid = lax.axis_index("s")
    wid = cid * NS + sid
    base = wid * EPW

    idx_descs = [
        pltpu.async_copy(dst_hbm.at[pl.ds(base + j * CH, CH)],
                         idx_v.at[j], sem_i)
        for j in range(NCH)
    ]
    rows_desc = pltpu.async_copy(msg_hbm.at[pl.ds(base, EPW)], rows_v, sem_r)

    def zbody(i, carry):
        stage_v[i, :] = jnp.zeros((H,), F32)
        return carry

    lax.fori_loop(0, VPT, zbody, 0)
    pltpu.sync_copy(stage_v, acc_sh.at[pl.ds(sid * VPT, VPT)])
    plsc.subcore_barrier()

    rows_desc.wait()
    descs = []
    for j in range(NCH):
        idx_descs[j].wait()
        descs.append(
            pltpu.async_copy(rows_v.at[pl.ds(j * CH, CH)],
                             acc_sh.at[idx_v.at[j]], sem_s, add=True))
    for d in descs:
        d.wait()
    plsc.subcore_barrier()

    pltpu.sync_copy(acc_sh.at[pl.ds(sid * VPT, VPT)], stage_v)
    pltpu.sync_copy(stage_v, out_hbm.at[cid, pl.ds(sid * VPT, VPT)])


def _mpnn_block(node, hidt, eft, src, dst, p, msg_blk, steps):
    w1 = p['eW1']
    b1c = p['eb1'].reshape(E_HID, 1)
    w2 = p['eW2']
    b2c = p['eb2'].reshape(H * H, 1)
    cbc = p['cb'].reshape(H, 1)
    wih = p['gWih']
    bihc = p['gbih'].reshape(3 * H, 1)
    whh = p['gWhh']
    bhhc = p['gbhh'].reshape(3 * H, 1)
    for _ in range(steps):
        g = _sc_gather(node, src)
        msgt = _msg(eft, g.T, w1, b1c, w2, b2c, msg_blk)
        parts = _sc_scatter(msgt.T, dst)
        hidt = _gru(parts.transpose(0, 2, 1), hidt, cbc, wih, bihc, whh, bhhc)
        node = hidt.T
    return hidt


def kernel(node_feats, edge_feats, params, edge_index):
    src = edge_index[0]
    dst = edge_index[1]

    e1w1t = params['encW1'].T
    e1b1 = params['encb1'].reshape(1, R // 2)
    e1w2t = params['encW2'].T
    e1b2 = params['encb2'].reshape(1, PR)

    x1 = node_feats.transpose(0, 1, 3, 2).reshape(6 * R * C, R)
    y = _lin2(x1, e1w1t, e1b1, e1w2t, e1b2, 8192)
    y = y.reshape(6, R, C, PR).transpose(0, 3, 2, 1)
    z = _lin2(y.reshape(6 * PR * C, R), e1w1t, e1b1, e1w2t, e1b2, 8192)
    xn = z.reshape(6, PR, C, PR).transpose(0, 3, 1, 2).reshape(V, C)

    pi = params['inp']
    h = _proj_rows(xn, pi['pW1'].T, pi['pb1'].reshape(1, H),
                   pi['pW2'].T, pi['pb2'].reshape(1, H))
    eft = edge_feats.T

    hidt = _mpnn_block(h, h.T, eft, src, dst, pi, 8192, 2)

    pp = params['proc0']
    h2t = _proj_t(hidt, pp['pW1'], pp['pb1'].reshape(H, 1),
                  pp['pW2'], pp['pb2'].reshape(H, 1))
    hidt = _mpnn_block(h2t.T, h2t, eft, src, dst, pp, 8192, 2)

    d1w1t = params['decW1'].T
    d1b1 = params['decb1'].reshape(1, R // 2)
    d1w2t = params['decW2'].T
    d1b2 = params['decb2'].reshape(1, R)

    xd = hidt.reshape(H, 6, PR, PR).transpose(1, 0, 2, 3)
    a = _lin2(xd.reshape(6 * H * PR, PR), d1w1t, d1b1, d1w2t, d1b2, 3072)
    a = a.reshape(6, H, PR, R).transpose(0, 1, 3, 2)
    b = _lin2(a.reshape(6 * H * R, PR), d1w1t, d1b1, d1w2t, d1b2, 6144)
    zt = b.reshape(6, H, R, R).transpose(1, 0, 3, 2)
    out_t = _head(zt.reshape(H, 6 * R * R),
                  params['dW1'], params['db1'].reshape(H // 2, 1),
                  params['dW2'], params['db2'].reshape(3, 1))
    return out_t.T


# speedup vs baseline: 4.1947x; 1.1391x over previous
import functools

import jax
import jax.numpy as jnp
from jax import lax
from jax.experimental import pallas as pl
from jax.experimental.pallas import tpu as pltpu
from jax.experimental.pallas import tpu_sc as plsc

R = 128
PR = R // 4
V = 6 * PR * PR
E = V * 16
H = 16
C = 128
E_HID = 64
F32 = jnp.float32
BF = jnp.bfloat16

NC = 2
NS = 16
NW = NC * NS
EPW = E // NW
CH = 128
NCH = EPW // CH
VPT = V // NS


def _bdot(a, b):
    return jnp.dot(a.astype(BF), b.astype(BF), preferred_element_type=F32)


def _rnd(x):
    return x.astype(BF).astype(F32)


def _lin2_body(x_ref, w1_ref, b1_ref, w2_ref, b2_ref, out_ref):
    y = _bdot(x_ref[...], w1_ref[...]) + b1_ref[...]
    out_ref[...] = _bdot(y, w2_ref[...]) + b2_ref[...]


def _lin2(x, w1t, b1, w2t, b2, blk):
    m = x.shape[0]
    return pl.pallas_call(
        _lin2_body,
        grid=(m // blk,),
        in_specs=[
            pl.BlockSpec((blk, x.shape[1]), lambda i: (i, 0)),
            pl.BlockSpec(w1t.shape, lambda i: (0, 0)),
            pl.BlockSpec(b1.shape, lambda i: (0, 0)),
            pl.BlockSpec(w2t.shape, lambda i: (0, 0)),
            pl.BlockSpec(b2.shape, lambda i: (0, 0)),
        ],
        out_specs=pl.BlockSpec((blk, w2t.shape[1]), lambda i: (i, 0)),
        out_shape=jax.ShapeDtypeStruct((m, w2t.shape[1]), F32),
    )(x, w1t, b1, w2t, b2)


def _proj_rows_body(x_ref, pw1_ref, pb1_ref, pw2_ref, pb2_ref, out_ref):
    h1 = jnp.maximum(_bdot(x_ref[...], pw1_ref[...]) + pb1_ref[...], 0.0)
    out_ref[...] = _bdot(h1, pw2_ref[...]) + pb2_ref[...]


def _proj_rows(x, pw1t, pb1, pw2t, pb2):
    return pl.pallas_call(
        _proj_rows_body,
        out_shape=jax.ShapeDtypeStruct((V, H), F32),
    )(x, pw1t, pb1, pw2t, pb2)


def _proj_t_body(x_ref, pw1_ref, pb1_ref, pw2_ref, pb2_ref, out_ref):
    h1 = jnp.maximum(_bdot(pw1_ref[...], x_ref[...]) + pb1_ref[...], 0.0)
    out_ref[...] = _bdot(pw2_ref[...], h1) + pb2_ref[...]


def _proj_t(xt, pw1, pb1c, pw2, pb2c):
    return pl.pallas_call(
        _proj_t_body,
        out_shape=jax.ShapeDtypeStruct((H, V), F32),
    )(xt, pw1, pb1c, pw2, pb2c)


def _msg_body(ef_ref, g_ref, w1_ref, b1_ref, w2_ref, b2_ref, out_ref):
    e1 = jnp.maximum(_bdot(w1_ref[...], ef_ref[...]) + b1_ref[...], 0.0)
    ew = _rnd(_bdot(w2_ref[...], e1) + b2_ref[...])
    g = _rnd(g_ref[...].T)
    acc = g[0:1, :] * ew[0:H, :]
    for i in range(1, H):
        acc = acc + g[i:i + 1, :] * ew[i * H:(i + 1) * H, :]
    out_ref[...] = acc.T


def _msg(eft, gt, w1, b1c, w2, b2c, blk):
    return pl.pallas_call(
        _msg_body,
        grid=(E // blk,),
        in_specs=[
            pl.BlockSpec((H, blk), lambda e: (0, e)),
            pl.BlockSpec((blk, H), lambda e: (e, 0)),
            pl.BlockSpec((E_HID, H), lambda e: (0, 0)),
            pl.BlockSpec((E_HID, 1), lambda e: (0, 0)),
            pl.BlockSpec((H * H, E_HID), lambda e: (0, 0)),
            pl.BlockSpec((H * H, 1), lambda e: (0, 0)),
        ],
        out_specs=pl.BlockSpec((blk, H), lambda e: (e, 0)),
        out_shape=jax.ShapeDtypeStruct((E, H), F32),
    )(eft, gt, w1, b1c, w2, b2c)


def _gru_body(p_ref, hid_ref, cb_ref, wih_ref, bih_ref, whh_ref, bhh_ref,
              out_ref):
    node = jnp.maximum(p_ref[0] + p_ref[1] + cb_ref[...], 0.0)
    gi = _bdot(wih_ref[...], node) + bih_ref[...]
    gh = _bdot(whh_ref[...], hid_ref[...]) + bhh_ref[...]
    r = jax.nn.sigmoid(gi[0:H, :] + gh[0:H, :])
    z = jax.nn.sigmoid(gi[H:2 * H, :] + gh[H:2 * H, :])
    n = jnp.tanh(gi[2 * H:3 * H, :] + r * gh[2 * H:3 * H, :])
    out_ref[...] = (1.0 - z) * n + z * hid_ref[...]


def _gru(partst, hidt, cbc, wih, bihc, whh, bhhc):
    return pl.pallas_call(
        _gru_body,
        out_shape=jax.ShapeDtypeStruct((H, V), F32),
    )(partst, hidt, cbc, wih, bihc, whh, bhhc)


def _head_body(z_ref, dw1_ref, db1_ref, dw2_ref, db2_ref, out_ref):
    h1 = jnp.maximum(_bdot(dw1_ref[...], z_ref[...]) + db1_ref[...], 0.0)
    out_ref[...] = _bdot(dw2_ref[...], h1) + db2_ref[...]


def _head(zt, dw1, db1c, dw2, db2c):
    return pl.pallas_call(
        _head_body,
        grid=(6,),
        in_specs=[
            pl.BlockSpec((H, R * R), lambda f: (0, f)),
            pl.BlockSpec((H // 2, H), lambda f: (0, 0)),
            pl.BlockSpec((H // 2, 1), lambda f: (0, 0)),
            pl.BlockSpec((3, H // 2), lambda f: (0, 0)),
            pl.BlockSpec((3, 1), lambda f: (0, 0)),
        ],
        out_specs=pl.BlockSpec((3, R * R), lambda f: (0, f)),
        out_shape=jax.ShapeDtypeStruct((3, 6 * R * R), F32),
    )(zt, dw1, db1c, dw2, db2c)


@functools.cache
def _sc_mesh():
    return plsc.VectorSubcoreMesh(core_axis_name="c", subcore_axis_name="s",
                                  num_cores=NC, num_subcores=NS)


@functools.cache
def _sc_gather_kernel():
    return pl.kernel(
        _sc_gather_body,
        out_type=jax.ShapeDtypeStruct((E, H), F32),
        mesh=_sc_mesh(),
        scratch_types=[
            pltpu.VMEM((NCH, CH), jnp.int32),
            pltpu.VMEM((EPW, H), F32),
            pltpu.SemaphoreType.DMA,
            pltpu.SemaphoreType.DMA,
        ],
        compiler_params=pltpu.CompilerParams(use_tc_tiling_on_sc=False),
    )


def _sc_gather(node, src):
    return _sc_gather_kernel()(node, src)


def _sc_gather_body(node_hbm, src_hbm, out_hbm, idx_v, rows_v, sem_i, sem_g):
    wid = lax.axis_index("c") * NS + lax.axis_index("s")
    base = wid * EPW

    idx_descs = [
        pltpu.async_copy(src_hbm.at[pl.ds(base + j * CH, CH)],
                         idx_v.at[j], sem_i)
        for j in range(NCH)
    ]
    descs = []
    for j in range(NCH):
        idx_descs[j].wait()
        descs.append(
            pltpu.async_copy(node_hbm.at[idx_v.at[j]],
                             rows_v.at[pl.ds(j * CH, CH)], sem_g))
    for d in descs:
        d.wait()
    pltpu.sync_copy(rows_v, out_hbm.at[pl.ds(base, EPW)])


@functools.cache
def _sc_scatter_kernel():
    return pl.kernel(
        _sc_scatter_body,
        out_type=jax.ShapeDtypeStruct((NC, V, H), F32),
        mesh=_sc_mesh(),
        scratch_types=[
            pltpu.VMEM((NCH, CH), jnp.int32),
            pltpu.VMEM((EPW, H), F32),
            pltpu.VMEM((VPT, H), F32),
            pltpu.VMEM_SHARED((V, H), F32),
            pltpu.SemaphoreType.DMA,
            pltpu.SemaphoreType.DMA,
            pltpu.SemaphoreType.DMA,
        ],
        compiler_params=pltpu.CompilerParams(use_tc_tiling_on_sc=False),
    )


def _sc_scatter(msg, dst):
    return _sc_scatter_kernel()(msg, dst)


def _sc_scatter_body(msg_hbm, dst_hbm, out_hbm, idx_v, rows_v, stage_v, acc_sh,
                     sem_i, sem_r, sem_s):
    cid = lax.axis_index("c")
    sid = lax.axis_index("s")
    wid = cid * NS + sid
    base = wid * EPW

    idx_descs = [
        pltpu.async_copy(dst_hbm.at[pl.ds(base + j * CH, CH)],
                         idx_v.at[j], sem_i)
        for j in range(NCH)
    ]
    rows_desc = pltpu.async_copy(msg_hbm.at[pl.ds(base, EPW)], rows_v, sem_r)

    def zbody(i, carry):
        stage_v[i, :] = jnp.zeros((H,), F32)
        return carry

    lax.fori_loop(0, VPT, zbody, 0)
    pltpu.sync_copy(stage_v, acc_sh.at[pl.ds(sid * VPT, VPT)])
    plsc.subcore_barrier()

    rows_desc.wait()
    descs = []
    for j in range(NCH):
        idx_descs[j].wait()
        descs.append(
            pltpu.async_copy(rows_v.at[pl.ds(j * CH, CH)],
                             acc_sh.at[idx_v.at[j]], sem_s, add=True))
    for d in descs:
        d.wait()
    plsc.subcore_barrier()

    pltpu.sync_copy(acc_sh.at[pl.ds(sid * VPT, VPT)], stage_v)
    pltpu.sync_copy(stage_v, out_hbm.at[cid, pl.ds(sid * VPT, VPT)])


def _mpnn_block(node, hidt, eft, src, dst, p, msg_blk, steps):
    w1 = p['eW1']
    b1c = p['eb1'].reshape(E_HID, 1)
    w2 = p['eW2']
    b2c = p['eb2'].reshape(H * H, 1)
    cbc = p['cb'].reshape(H, 1)
    wih = p['gWih']
    bihc = p['gbih'].reshape(3 * H, 1)
    whh = p['gWhh']
    bhhc = p['gbhh'].reshape(3 * H, 1)
    for _ in range(steps):
        g = _sc_gather(node, src)
        msg = _msg(eft, g, w1, b1c, w2, b2c, msg_blk)
        parts = _sc_scatter(msg, dst)
        hidt = _gru(parts.transpose(0, 2, 1), hidt, cbc, wih, bihc, whh, bhhc)
        node = hidt.T
    return hidt


def kernel(node_feats, edge_feats, params, edge_index):
    src = edge_index[0]
    dst = edge_index[1]

    e1w1t = params['encW1'].T
    e1b1 = params['encb1'].reshape(1, R // 2)
    e1w2t = params['encW2'].T
    e1b2 = params['encb2'].reshape(1, PR)

    x1 = node_feats.transpose(0, 1, 3, 2).reshape(6 * R * C, R)
    y = _lin2(x1, e1w1t, e1b1, e1w2t, e1b2, 8192)
    y = y.reshape(6, R, C, PR).transpose(0, 3, 2, 1)
    z = _lin2(y.reshape(6 * PR * C, R), e1w1t, e1b1, e1w2t, e1b2, 8192)
    xn = z.reshape(6, PR, C, PR).transpose(0, 3, 1, 2).reshape(V, C)

    pi = params['inp']
    h = _proj_rows(xn, pi['pW1'].T, pi['pb1'].reshape(1, H),
                   pi['pW2'].T, pi['pb2'].reshape(1, H))
    eft = edge_feats.T

    hidt = _mpnn_block(h, h.T, eft, src, dst, pi, 8192, 2)

    pp = params['proc0']
    h2t = _proj_t(hidt, pp['pW1'], pp['pb1'].reshape(H, 1),
                  pp['pW2'], pp['pb2'].reshape(H, 1))
    hidt = _mpnn_block(h2t.T, h2t, eft, src, dst, pp, 8192, 2)

    d1w1t = params['decW1'].T
    d1b1 = params['decb1'].reshape(1, R // 2)
    d1w2t = params['decW2'].T
    d1b2 = params['decb2'].reshape(1, R)

    xd = hidt.reshape(H, 6, PR, PR).transpose(1, 0, 2, 3)
    a = _lin2(xd.reshape(6 * H * PR, PR), d1w1t, d1b1, d1w2t, d1b2, 3072)
    a = a.reshape(6, H, PR, R).transpose(0, 1, 3, 2)
    b = _lin2(a.reshape(6 * H * R, PR), d1w1t, d1b1, d1w2t, d1b2, 6144)
    zt = b.reshape(6, H, R, R).transpose(1, 0, 3, 2)
    out_t = _head(zt.reshape(H, 6 * R * R),
                  params['dW1'], params['db1'].reshape(H // 2, 1),
                  params['dW2'], params['db2'].reshape(3, 1))
    return out_t.T


# speedup vs baseline: 4.5034x; 1.0736x over previous
import functools

import jax
import jax.numpy as jnp
from jax import lax
from jax.experimental import pallas as pl
from jax.experimental.pallas import tpu as pltpu
from jax.experimental.pallas import tpu_sc as plsc

R = 128
PR = R // 4
V = 6 * PR * PR
E = V * 16
H = 16
C = 128
E_HID = 64
F32 = jnp.float32
BF = jnp.bfloat16

NC = 2
NS = 16
NW = NC * NS
EPW = E // NW
CH = 128
NCH = EPW // CH
VPT = V // NS


def _bdot(a, b):
    return jnp.dot(a.astype(BF), b.astype(BF), preferred_element_type=F32)


def _rnd(x):
    return x.astype(BF).astype(F32)


def _lin2_body(x_ref, w1_ref, b1_ref, w2_ref, b2_ref, out_ref):
    y = _bdot(x_ref[...], w1_ref[...]) + b1_ref[...]
    out_ref[...] = _bdot(y, w2_ref[...]) + b2_ref[...]


def _lin2(x, w1t, b1, w2t, b2, blk):
    m = x.shape[0]
    return pl.pallas_call(
        _lin2_body,
        grid=(m // blk,),
        in_specs=[
            pl.BlockSpec((blk, x.shape[1]), lambda i: (i, 0)),
            pl.BlockSpec(w1t.shape, lambda i: (0, 0)),
            pl.BlockSpec(b1.shape, lambda i: (0, 0)),
            pl.BlockSpec(w2t.shape, lambda i: (0, 0)),
            pl.BlockSpec(b2.shape, lambda i: (0, 0)),
        ],
        out_specs=pl.BlockSpec((blk, w2t.shape[1]), lambda i: (i, 0)),
        out_shape=jax.ShapeDtypeStruct((m, w2t.shape[1]), F32),
    )(x, w1t, b1, w2t, b2)


def _enc1_body(x_ref, w1_ref, b1_ref, w2_ref, b2_ref, out_ref):
    xt = jnp.swapaxes(x_ref[0], 1, 2)
    x2 = xt.reshape(xt.shape[0] * xt.shape[1], xt.shape[2])
    y = _bdot(x2, w1_ref[...]) + b1_ref[...]
    out_ref[...] = _bdot(y, w2_ref[...]) + b2_ref[...]


def _enc1(x4, w1t, b1, w2t, b2, bi):
    nb = R // bi
    return pl.pallas_call(
        _enc1_body,
        grid=(6, nb),
        in_specs=[
            pl.BlockSpec((1, bi, R, C), lambda f, ib: (f, ib, 0, 0)),
            pl.BlockSpec(w1t.shape, lambda f, ib: (0, 0)),
            pl.BlockSpec(b1.shape, lambda f, ib: (0, 0)),
            pl.BlockSpec(w2t.shape, lambda f, ib: (0, 0)),
            pl.BlockSpec(b2.shape, lambda f, ib: (0, 0)),
        ],
        out_specs=pl.BlockSpec((bi * C, PR), lambda f, ib: (f * (R // bi) + ib, 0)),
        out_shape=jax.ShapeDtypeStruct((6 * R * C, PR), F32),
    )(x4, w1t, b1, w2t, b2)


def _lin2s_body(x_ref, w1_ref, b1_ref, w2_ref, b2_ref, out_ref):
    xt = jnp.swapaxes(x_ref[...], 1, 2)
    x2 = xt.reshape(xt.shape[0] * xt.shape[1], xt.shape[2])
    y = _bdot(x2, w1_ref[...]) + b1_ref[...]
    out_ref[...] = _bdot(y, w2_ref[...]) + b2_ref[...]


def _lin2s(x3, w1t, b1, w2t, b2, bfc):
    g_, kdim, ndim = x3.shape
    return pl.pallas_call(
        _lin2s_body,
        grid=(g_ // bfc,),
        in_specs=[
            pl.BlockSpec((bfc, kdim, ndim), lambda i: (i, 0, 0)),
            pl.BlockSpec(w1t.shape, lambda i: (0, 0)),
            pl.BlockSpec(b1.shape, lambda i: (0, 0)),
            pl.BlockSpec(w2t.shape, lambda i: (0, 0)),
            pl.BlockSpec(b2.shape, lambda i: (0, 0)),
        ],
        out_specs=pl.BlockSpec((bfc * ndim, w2t.shape[1]), lambda i: (i, 0)),
        out_shape=jax.ShapeDtypeStruct((g_ * ndim, w2t.shape[1]), F32),
    )(x3, w1t, b1, w2t, b2)


def _proj_rows_body(x_ref, pw1_ref, pb1_ref, pw2_ref, pb2_ref, out_ref):
    h1 = jnp.maximum(_bdot(x_ref[...], pw1_ref[...]) + pb1_ref[...], 0.0)
    out_ref[...] = _bdot(h1, pw2_ref[...]) + pb2_ref[...]


def _proj_rows(x, pw1t, pb1, pw2t, pb2):
    return pl.pallas_call(
        _proj_rows_body,
        out_shape=jax.ShapeDtypeStruct((V, H), F32),
    )(x, pw1t, pb1, pw2t, pb2)


def _proj_t_body(x_ref, pw1_ref, pb1_ref, pw2_ref, pb2_ref, out_ref):
    h1 = jnp.maximum(_bdot(pw1_ref[...], x_ref[...]) + pb1_ref[...], 0.0)
    out_ref[...] = _bdot(pw2_ref[...], h1) + pb2_ref[...]


def _proj_t(xt, pw1, pb1c, pw2, pb2c):
    return pl.pallas_call(
        _proj_t_body,
        out_shape=jax.ShapeDtypeStruct((H, V), F32),
    )(xt, pw1, pb1c, pw2, pb2c)


def _msg_body(ef_ref, g_ref, w1_ref, b1_ref, w2_ref, b2_ref, out_ref):
    e1 = jnp.maximum(_bdot(w1_ref[...], ef_ref[...]) + b1_ref[...], 0.0)
    ew = _rnd(_bdot(w2_ref[...], e1) + b2_ref[...])
    g = _rnd(g_ref[...].T)
    acc = g[0:1, :] * ew[0:H, :]
    for i in range(1, H):
        acc = acc + g[i:i + 1, :] * ew[i * H:(i + 1) * H, :]
    out_ref[...] = acc.T


def _msg(eft, gt, w1, b1c, w2, b2c, blk):
    return pl.pallas_call(
        _msg_body,
        grid=(E // blk,),
        in_specs=[
            pl.BlockSpec((H, blk), lambda e: (0, e)),
            pl.BlockSpec((blk, H), lambda e: (e, 0)),
            pl.BlockSpec((E_HID, H), lambda e: (0, 0)),
            pl.BlockSpec((E_HID, 1), lambda e: (0, 0)),
            pl.BlockSpec((H * H, E_HID), lambda e: (0, 0)),
            pl.BlockSpec((H * H, 1), lambda e: (0, 0)),
        ],
        out_specs=pl.BlockSpec((blk, H), lambda e: (e, 0)),
        out_shape=jax.ShapeDtypeStruct((E, H), F32),
    )(eft, gt, w1, b1c, w2, b2c)


def _gru_body(p_ref, hid_ref, cb_ref, wih_ref, bih_ref, whh_ref, bhh_ref,
              out_ref):
    node = jnp.maximum(p_ref[0] + p_ref[1] + cb_ref[...], 0.0)
    gi = _bdot(wih_ref[...], node) + bih_ref[...]
    gh = _bdot(whh_ref[...], hid_ref[...]) + bhh_ref[...]
    r = jax.nn.sigmoid(gi[0:H, :] + gh[0:H, :])
    z = jax.nn.sigmoid(gi[H:2 * H, :] + gh[H:2 * H, :])
    n = jnp.tanh(gi[2 * H:3 * H, :] + r * gh[2 * H:3 * H, :])
    out_ref[...] = (1.0 - z) * n + z * hid_ref[...]


def _gru(partst, hidt, cbc, wih, bihc, whh, bhhc):
    return pl.pallas_call(
        _gru_body,
        out_shape=jax.ShapeDtypeStruct((H, V), F32),
    )(partst, hidt, cbc, wih, bihc, whh, bhhc)


def _head_body(z_ref, dw1_ref, db1_ref, dw2_ref, db2_ref, out_ref):
    z3 = jnp.swapaxes(z_ref[0], 1, 2)
    z2 = z3.reshape(H, R * R)
    h1 = jnp.maximum(_bdot(dw1_ref[...], z2) + db1_ref[...], 0.0)
    out_ref[...] = _bdot(dw2_ref[...], h1) + db2_ref[...]


def _head(b4, dw1, db1c, dw2, db2c):
    return pl.pallas_call(
        _head_body,
        grid=(6,),
        in_specs=[
            pl.BlockSpec((1, H, R, R), lambda f: (f, 0, 0, 0)),
            pl.BlockSpec((H // 2, H), lambda f: (0, 0)),
            pl.BlockSpec((H // 2, 1), lambda f: (0, 0)),
            pl.BlockSpec((3, H // 2), lambda f: (0, 0)),
            pl.BlockSpec((3, 1), lambda f: (0, 0)),
        ],
        out_specs=pl.BlockSpec((3, R * R), lambda f: (0, f)),
        out_shape=jax.ShapeDtypeStruct((3, 6 * R * R), F32),
    )(b4, dw1, db1c, dw2, db2c)


@functools.cache
def _sc_mesh():
    return plsc.VectorSubcoreMesh(core_axis_name="c", subcore_axis_name="s",
                                  num_cores=NC, num_subcores=NS)


@functools.cache
def _sc_gather_kernel():
    return pl.kernel(
        _sc_gather_body,
        out_type=jax.ShapeDtypeStruct((E, H), F32),
        mesh=_sc_mesh(),
        scratch_types=[
            pltpu.VMEM((NCH, CH), jnp.int32),
            pltpu.VMEM((EPW, H), F32),
            pltpu.SemaphoreType.DMA,
            pltpu.SemaphoreType.DMA,
        ],
        compiler_params=pltpu.CompilerParams(use_tc_tiling_on_sc=False),
    )


def _sc_gather(node, src):
    return _sc_gather_kernel()(node, src)


def _sc_gather_body(node_hbm, src_hbm, out_hbm, idx_v, rows_v, sem_i, sem_g):
    wid = lax.axis_index("c") * NS + lax.axis_index("s")
    base = wid * EPW

    idx_descs = [
        pltpu.async_copy(src_hbm.at[pl.ds(base + j * CH, CH)],
                         idx_v.at[j], sem_i)
        for j in range(NCH)
    ]
    descs = []
    for j in range(NCH):
        idx_descs[j].wait()
        descs.append(
            pltpu.async_copy(node_hbm.at[idx_v.at[j]],
                             rows_v.at[pl.ds(j * CH, CH)], sem_g))
    for d in descs:
        d.wait()
    pltpu.sync_copy(rows_v, out_hbm.at[pl.ds(base, EPW)])


@functools.cache
def _sc_scatter_kernel():
    return pl.kernel(
        _sc_scatter_body,
        out_type=jax.ShapeDtypeStruct((NC, V, H), F32),
        mesh=_sc_mesh(),
        scratch_types=[
            pltpu.VMEM((NCH, CH), jnp.int32),
            pltpu.VMEM((EPW, H), F32),
            pltpu.VMEM((VPT, H), F32),
            pltpu.VMEM_SHARED((V, H), F32),
            pltpu.SemaphoreType.DMA,
            pltpu.SemaphoreType.DMA,
            pltpu.SemaphoreType.DMA,
        ],
        compiler_params=pltpu.CompilerParams(use_tc_tiling_on_sc=False),
    )


def _sc_scatter(msg, dst):
    return _sc_scatter_kernel()(msg, dst)


def _sc_scatter_body(msg_hbm, dst_hbm, out_hbm, idx_v, rows_v, stage_v, acc_sh,
                     sem_i, sem_r, sem_s):
    cid = lax.axis_index("c")
    sid = lax.axis_index("s")
    wid = cid * NS + sid
    base = wid * EPW

    idx_descs = [
        pltpu.async_copy(dst_hbm.at[pl.ds(base + j * CH, CH)],
                         idx_v.at[j], sem_i)
        for j in range(NCH)
    ]
    rows_desc = pltpu.async_copy(msg_hbm.at[pl.ds(base, EPW)], rows_v, sem_r)

    def zbody(i, carry):
        stage_v[i, :] = jnp.zeros((H,), F32)
        return carry

    lax.fori_loop(0, VPT, zbody, 0)
    pltpu.sync_copy(stage_v, acc_sh.at[pl.ds(sid * VPT, VPT)])
    plsc.subcore_barrier()

    rows_desc.wait()
    descs = []
    for j in range(NCH):
        idx_descs[j].wait()
        descs.append(
            pltpu.async_copy(rows_v.at[pl.ds(j * CH, CH)],
                             acc_sh.at[idx_v.at[j]], sem_s, add=True))
    for d in descs:
        d.wait()
    plsc.subcore_barrier()

    pltpu.sync_copy(acc_sh.at[pl.ds(sid * VPT, VPT)], stage_v)
    pltpu.sync_copy(stage_v, out_hbm.at[cid, pl.ds(sid * VPT, VPT)])


def _mpnn_block(node, hidt, eft, src, dst, p, msg_blk, steps):
    w1 = p['eW1']
    b1c = p['eb1'].reshape(E_HID, 1)
    w2 = p['eW2']
    b2c = p['eb2'].reshape(H * H, 1)
    cbc = p['cb'].reshape(H, 1)
    wih = p['gWih']
    bihc = p['gbih'].reshape(3 * H, 1)
    whh = p['gWhh']
    bhhc = p['gbhh'].reshape(3 * H, 1)
    for _ in range(steps):
        g = _sc_gather(node, src)
        msg = _msg(eft, g, w1, b1c, w2, b2c, msg_blk)
        parts = _sc_scatter(msg, dst)
        hidt = _gru(parts.transpose(0, 2, 1), hidt, cbc, wih, bihc, whh, bhhc)
        node = hidt.T
    return hidt


def kernel(node_feats, edge_feats, params, edge_index):
    src = edge_index[0]
    dst = edge_index[1]

    e1w1t = params['encW1'].T
    e1b1 = params['encb1'].reshape(1, R // 2)
    e1w2t = params['encW2'].T
    e1b2 = params['encb2'].reshape(1, PR)

    y = _enc1(node_feats, e1w1t, e1b1, e1w2t, e1b2, 32)
    y = y.reshape(6, R, C, PR).transpose(0, 3, 2, 1)
    z = _lin2(y.reshape(6 * PR * C, R), e1w1t, e1b1, e1w2t, e1b2, 8192)
    xn = z.reshape(6, PR, C, PR).transpose(0, 3, 1, 2).reshape(V, C)

    pi = params['inp']
    h = _proj_rows(xn, pi['pW1'].T, pi['pb1'].reshape(1, H),
                   pi['pW2'].T, pi['pb2'].reshape(1, H))
    eft = edge_feats.T

    hidt = _mpnn_block(h, h.T, eft, src, dst, pi, 8192, 2)

    pp = params['proc0']
    h2t = _proj_t(hidt, pp['pW1'], pp['pb1'].reshape(H, 1),
                  pp['pW2'], pp['pb2'].reshape(H, 1))
    hidt = _mpnn_block(h2t.T, h2t, eft, src, dst, pp, 8192, 2)

    d1w1t = params['decW1'].T
    d1b1 = params['decb1'].reshape(1, R // 2)
    d1w2t = params['decW2'].T
    d1b2 = params['decb2'].reshape(1, R)

    xd = hidt.reshape(H, 6, PR, PR).transpose(1, 0, 2, 3)
    a = _lin2(xd.reshape(6 * H * PR, PR), d1w1t, d1b1, d1w2t, d1b2, 3072)
    b = _lin2s(a.reshape(6 * H, PR, R), d1w1t, d1b1, d1w2t, d1b2, 16)
    out_t = _head(b.reshape(6, H, R, R),
                  params['dW1'], params['db1'].reshape(H // 2, 1),
                  params['dW2'], params['db2'].reshape(3, 1))
    return out_t.T
